# Initial kernel scaffold; baseline (speedup 1.0000x reference)
import jax
import jax.numpy as jnp
from jax.experimental import pallas as pl


def kernel(species, energies, atom_index12, distances, exp_prefactor, distances_factor):
    raise NotImplementedError("write your pallas kernel here")


# speedup vs baseline: 385.2024x; 385.2024x over previous
import dataclasses
import functools

import jax
import jax.numpy as jnp
from jax import lax
from jax.experimental import pallas as pl
from jax.experimental.pallas import tpu as pltpu
from jax.experimental.pallas import tpu_sc as plsc

ANGSTROM2BOHR = 1.8897261258369282
CUTOFF = 5.2 * ANGSTROM2BOHR
LANES = 16
NUM_WORKERS = 32
CHUNK = 2000


def _srb_sc_kernel(n_edges, n_flat_atoms, num_atoms, n_bins):
  grid = n_edges // CHUNK
  mesh = plsc.VectorSubcoreMesh(core_axis_name="c", subcore_axis_name="s")

  cp = pltpu.CompilerParams()
  if "needs_layout_passes" in pltpu.CompilerParams.__dataclass_fields__:
    cp = dataclasses.replace(cp, needs_layout_passes=False)

  @functools.partial(
      pl.kernel,
      out_type=jax.ShapeDtypeStruct((NUM_WORKERS, n_bins), jnp.float32),
      mesh=mesh,
      compiler_params=cp,
      scratch_types=[
          pltpu.VMEM((n_flat_atoms,), jnp.int32),
          pltpu.VMEM((LANES,), jnp.float32),
          pltpu.VMEM((LANES,), jnp.float32),
          pltpu.VMEM((n_bins,), jnp.float32),
      ],
  )
  def kern(sp_hbm, a0_hbm, a1_hbm, d_hbm, pref_hbm, dfac_hbm, out_hbm,
           sp_v, pref_v, dfac_v, acc_v):
    pltpu.sync_copy(sp_hbm, sp_v)
    pltpu.sync_copy(pref_hbm, pref_v)
    pltpu.sync_copy(dfac_hbm, dfac_v)

    @pl.loop(0, n_bins, step=LANES)
    def _(i):
      acc_v[pl.ds(i, LANES)] = jnp.zeros((LANES,), jnp.float32)

    inv_cutoff = jnp.float32(1.0 / CUTOFF)
    a2b = jnp.float32(ANGSTROM2BOHR)

    def chunk_body(a0_v, a1_v, d_v):
      @pl.loop(0, CHUNK, step=LANES)
      def _(c):
        i0 = a0_v[pl.ds(c, LANES)]
        i1 = a1_v[pl.ds(c, LANES)]
        dist = d_v[pl.ds(c, LANES)]
        s1 = plsc.load_gather(sp_v, [i0])
        s2 = plsc.load_gather(sp_v, [i1])
        k = s1 * 4 + s2
        pref = plsc.load_gather(pref_v, [k])
        dfac = plsc.load_gather(dfac_v, [k])
        d = dist * a2b
        x = d * inv_cutoff
        x2 = x * x
        inside = x < 1.0
        earg = dfac * d + (1.0 - 1.0 / (1.0 - x2))
        val = pref * jnp.exp(earg)
        val = jnp.where(inside, val, jnp.float32(0.0))
        mol = i0 // num_atoms
        plsc.addupdate_scatter(acc_v, [mol], val)

    pltpu.emit_pipeline(
        chunk_body,
        grid=(grid,),
        in_specs=[
            pl.BlockSpec((CHUNK,), lambda i: (i,)),
            pl.BlockSpec((CHUNK,), lambda i: (i,)),
            pl.BlockSpec((CHUNK,), lambda i: (i,)),
        ],
        out_specs=[],
        core_axis_name=("c", "s"),
        dimension_semantics=(pltpu.PARALLEL,),
    )(a0_hbm, a1_hbm, d_hbm)

    wid = lax.axis_index("s") * 2 + lax.axis_index("c")
    pltpu.sync_copy(acc_v, out_hbm.at[wid])

  return kern


def kernel(species, energies, atom_index12, distances, exp_prefactor,
           distances_factor):
  n_mol = energies.shape[0]
  num_atoms = species.shape[1]
  n_edges = distances.shape[0]
  n_bins = 1024

  flat_species = species.reshape(-1).astype(jnp.int32)
  a0 = atom_index12[0].astype(jnp.int32)
  a1 = atom_index12[1].astype(jnp.int32)
  pref16 = exp_prefactor.reshape(-1).astype(jnp.float32)
  dfac16 = distances_factor.reshape(-1).astype(jnp.float32)

  kern = _srb_sc_kernel(n_edges, flat_species.shape[0], num_atoms, n_bins)
  partial = kern(flat_species, a0, a1, distances.astype(jnp.float32),
                 pref16, dfac16)
  energies_out = energies + partial.sum(axis=0)[:n_mol]
  return species, energies_out


# speedup vs baseline: 580.0089x; 1.5057x over previous
import dataclasses
import functools

import jax
import jax.numpy as jnp
from jax import lax
from jax.experimental import pallas as pl
from jax.experimental.pallas import tpu as pltpu
from jax.experimental.pallas import tpu_sc as plsc

ANGSTROM2BOHR = 1.8897261258369282
CUTOFF = 5.2 * ANGSTROM2BOHR
LANES = 16
NUM_WORKERS = 32
CHUNK = 2000


def _srb_sc_kernel(n_edges, n_flat_atoms, num_atoms, n_bins):
  grid = n_edges // CHUNK
  mesh = plsc.VectorSubcoreMesh(core_axis_name="c", subcore_axis_name="s")

  cp = pltpu.CompilerParams()
  if "needs_layout_passes" in pltpu.CompilerParams.__dataclass_fields__:
    cp = dataclasses.replace(cp, needs_layout_passes=False)

  @functools.partial(
      pl.kernel,
      out_type=jax.ShapeDtypeStruct((NUM_WORKERS, n_bins), jnp.float32),
      mesh=mesh,
      compiler_params=cp,
      scratch_types=[
          pltpu.VMEM((n_flat_atoms,), jnp.int32),
          pltpu.VMEM((LANES,), jnp.float32),
          pltpu.VMEM((LANES,), jnp.float32),
          pltpu.VMEM((n_bins,), jnp.float32),
      ],
  )
  def kern(sp_hbm, a0_hbm, a1_hbm, d_hbm, pref_hbm, dfac_hbm, out_hbm,
           sp_v, pref_v, dfac_v, acc_v):
    pltpu.sync_copy(sp_hbm, sp_v)
    pltpu.sync_copy(pref_hbm, pref_v)
    pltpu.sync_copy(dfac_hbm, dfac_v)

    @pl.loop(0, n_bins, step=LANES)
    def _(i):
      acc_v[pl.ds(i, LANES)] = jnp.zeros((LANES,), jnp.float32)

    inv_cutoff = jnp.float32(1.0 / CUTOFF)
    a2b = jnp.float32(ANGSTROM2BOHR)
    inv_atoms = jnp.float32(1.0 / num_atoms)

    def chunk_body(a0_v, a1_v, d_v):
      @pl.loop(0, CHUNK, step=LANES)
      def _(c):
        i0 = a0_v[pl.ds(c, LANES)]
        i1 = a1_v[pl.ds(c, LANES)]
        dist = d_v[pl.ds(c, LANES)]
        s1 = plsc.load_gather(sp_v, [i0])
        s2 = plsc.load_gather(sp_v, [i1])
        k = s1 * 4 + s2
        pref = plsc.load_gather(pref_v, [k])
        dfac = plsc.load_gather(dfac_v, [k])
        d = dist * a2b
        x = d * inv_cutoff
        x2 = x * x
        inside = x < 1.0
        earg = dfac * d + (1.0 - 1.0 / (1.0 - x2))
        val = pref * jnp.exp(earg)
        val = jnp.where(inside, val, jnp.float32(0.0))
        mol = ((i0.astype(jnp.float32) + 0.5) * inv_atoms).astype(jnp.int32)
        plsc.addupdate_scatter(acc_v, [mol], val)

    pltpu.emit_pipeline(
        chunk_body,
        grid=(grid,),
        in_specs=[
            pl.BlockSpec((CHUNK,), lambda i: (i,)),
            pl.BlockSpec((CHUNK,), lambda i: (i,)),
            pl.BlockSpec((CHUNK,), lambda i: (i,)),
        ],
        out_specs=[],
        core_axis_name=("c", "s"),
        dimension_semantics=(pltpu.PARALLEL,),
    )(a0_hbm, a1_hbm, d_hbm)

    wid = lax.axis_index("s") * 2 + lax.axis_index("c")
    pltpu.sync_copy(acc_v, out_hbm.at[wid])

  return kern


def kernel(species, energies, atom_index12, distances, exp_prefactor,
           distances_factor):
  n_mol = energies.shape[0]
  num_atoms = species.shape[1]
  n_edges = distances.shape[0]
  n_bins = 1024

  flat_species = species.reshape(-1).astype(jnp.int32)
  a0 = atom_index12[0].astype(jnp.int32)
  a1 = atom_index12[1].astype(jnp.int32)
  pref16 = exp_prefactor.reshape(-1).astype(jnp.float32)
  dfac16 = distances_factor.reshape(-1).astype(jnp.float32)

  kern = _srb_sc_kernel(n_edges, flat_species.shape[0], num_atoms, n_bins)
  partial = kern(flat_species, a0, a1, distances.astype(jnp.float32),
                 pref16, dfac16)
  energies_out = energies + partial.sum(axis=0)[:n_mol]
  return species, energies_out


# speedup vs baseline: 1304.8518x; 2.2497x over previous
import dataclasses
import functools

import jax
import jax.numpy as jnp
from jax import lax
from jax.experimental import pallas as pl
from jax.experimental.pallas import tpu as pltpu
from jax.experimental.pallas import tpu_sc as plsc

ANGSTROM2BOHR = 1.8897261258369282
CUTOFF = 5.2 * ANGSTROM2BOHR
LANES = 16
NUM_WORKERS = 32
CHUNK = 2000


def _srb_sc_kernel(n_edges, n_flat_atoms, num_atoms, n_bins):
  grid = n_edges // CHUNK
  mesh = plsc.VectorSubcoreMesh(core_axis_name="c", subcore_axis_name="s")

  cp = pltpu.CompilerParams()
  if "needs_layout_passes" in pltpu.CompilerParams.__dataclass_fields__:
    cp = dataclasses.replace(cp, needs_layout_passes=False)

  @functools.partial(
      pl.kernel,
      out_type=jax.ShapeDtypeStruct((NUM_WORKERS, n_bins), jnp.float32),
      mesh=mesh,
      compiler_params=cp,
      scratch_types=[
          pltpu.VMEM((n_flat_atoms,), jnp.int32),
          pltpu.VMEM((LANES,), jnp.float32),
          pltpu.VMEM((LANES,), jnp.float32),
          pltpu.VMEM((n_bins,), jnp.float32),
      ],
  )
  def kern(sp_hbm, a0_hbm, a1_hbm, d_hbm, pref_hbm, dfac_hbm, out_hbm,
           sp_v, pref_v, dfac_v, acc_v):
    pltpu.sync_copy(sp_hbm, sp_v)
    pltpu.sync_copy(pref_hbm, pref_v)
    pltpu.sync_copy(dfac_hbm, dfac_v)

    @pl.loop(0, n_bins, step=LANES)
    def _(i):
      acc_v[pl.ds(i, LANES)] = jnp.zeros((LANES,), jnp.float32)

    inv_cutoff = jnp.float32(1.0 / CUTOFF)
    a2b = jnp.float32(ANGSTROM2BOHR)
    inv_atoms = jnp.float32(1.0 / num_atoms)

    def chunk_body(a0_v, a1_v, d_v):
      @plsc.parallel_loop(0, CHUNK, step=LANES, unroll=4)
      def _(c):
        i0 = a0_v[pl.ds(c, LANES)]
        i1 = a1_v[pl.ds(c, LANES)]
        dist = d_v[pl.ds(c, LANES)]
        s1 = plsc.load_gather(sp_v, [i0])
        s2 = plsc.load_gather(sp_v, [i1])
        k = s1 * 4 + s2
        pref = plsc.load_gather(pref_v, [k])
        dfac = plsc.load_gather(dfac_v, [k])
        d = dist * a2b
        x = d * inv_cutoff
        x2 = x * x
        inside = x < 1.0
        earg = dfac * d + (1.0 - 1.0 / (1.0 - x2))
        val = pref * jnp.exp(earg)
        val = jnp.where(inside, val, jnp.float32(0.0))
        mol = ((i0.astype(jnp.float32) + 0.5) * inv_atoms).astype(jnp.int32)
        plsc.addupdate_scatter(acc_v, [mol], val)

    pltpu.emit_pipeline(
        chunk_body,
        grid=(grid,),
        in_specs=[
            pl.BlockSpec((CHUNK,), lambda i: (i,)),
            pl.BlockSpec((CHUNK,), lambda i: (i,)),
            pl.BlockSpec((CHUNK,), lambda i: (i,)),
        ],
        out_specs=[],
        core_axis_name=("c", "s"),
        dimension_semantics=(pltpu.PARALLEL,),
    )(a0_hbm, a1_hbm, d_hbm)

    wid = lax.axis_index("s") * 2 + lax.axis_index("c")
    pltpu.sync_copy(acc_v, out_hbm.at[wid])

  return kern


def kernel(species, energies, atom_index12, distances, exp_prefactor,
           distances_factor):
  n_mol = energies.shape[0]
  num_atoms = species.shape[1]
  n_edges = distances.shape[0]
  n_bins = 1024

  flat_species = species.reshape(-1).astype(jnp.int32)
  a0 = atom_index12[0].astype(jnp.int32)
  a1 = atom_index12[1].astype(jnp.int32)
  pref16 = exp_prefactor.reshape(-1).astype(jnp.float32)
  dfac16 = distances_factor.reshape(-1).astype(jnp.float32)

  kern = _srb_sc_kernel(n_edges, flat_species.shape[0], num_atoms, n_bins)
  partial = kern(flat_species, a0, a1, distances.astype(jnp.float32),
                 pref16, dfac16)
  energies_out = energies + partial.sum(axis=0)[:n_mol]
  return species, energies_out


# speedup vs baseline: 1388.2908x; 1.0639x over previous
import dataclasses
import functools

import jax
import jax.numpy as jnp
import numpy as np
from jax import lax
from jax.experimental import pallas as pl
from jax.experimental.pallas import tpu as pltpu
from jax.experimental.pallas import tpu_sc as plsc

ANGSTROM2BOHR = 1.8897261258369282
CUTOFF = 5.2 * ANGSTROM2BOHR
LANES = 16
NUM_WORKERS = 32
CHUNK = 2000


def _srb_sc_kernel(n_edges, n_flat_atoms, num_atoms, n_bins):
  grid = n_edges // CHUNK
  mesh = plsc.VectorSubcoreMesh(core_axis_name="c", subcore_axis_name="s")

  cp = pltpu.CompilerParams()
  if "needs_layout_passes" in pltpu.CompilerParams.__dataclass_fields__:
    cp = dataclasses.replace(cp, needs_layout_passes=False)

  @functools.partial(
      pl.kernel,
      out_type=jax.ShapeDtypeStruct((NUM_WORKERS, n_bins), jnp.float32),
      mesh=mesh,
      compiler_params=cp,
      scratch_types=[
          pltpu.VMEM((n_flat_atoms,), jnp.int32),
          pltpu.VMEM((LANES,), jnp.float32),
          pltpu.VMEM((LANES,), jnp.float32),
          pltpu.VMEM((n_bins,), jnp.float32),
      ],
  )
  def kern(sp_hbm, a12_hbm, d_hbm, pref_hbm, dfac_hbm, out_hbm,
           sp_v, pref_v, dfac_v, acc_v):
    pltpu.sync_copy(sp_hbm, sp_v)
    pltpu.sync_copy(pref_hbm, pref_v)
    pltpu.sync_copy(dfac_hbm, dfac_v)

    @pl.loop(0, n_bins, step=LANES)
    def _(i):
      acc_v[pl.ds(i, LANES)] = jnp.zeros((LANES,), jnp.float32)

    inv_cutoff = jnp.float32(ANGSTROM2BOHR / CUTOFF)
    inv_atoms = jnp.float32(
        float(np.nextafter(np.float32(1.0 / num_atoms), np.float32(2.0))))

    def chunk_body(a0_v, a1_v, d_v):
      @plsc.parallel_loop(0, CHUNK, step=LANES, unroll=8)
      def _(c):
        i0 = a0_v[pl.ds(c, LANES)]
        i1 = a1_v[pl.ds(c, LANES)]
        dist = d_v[pl.ds(c, LANES)]
        s1 = plsc.load_gather(sp_v, [i0])
        s2 = plsc.load_gather(sp_v, [i1])
        k = s1 * 4 + s2
        pref = plsc.load_gather(pref_v, [k])
        dfac = plsc.load_gather(dfac_v, [k])
        x = dist * inv_cutoff
        x2 = x * x
        inside = x < 1.0
        earg = dfac * dist + (1.0 - 1.0 / (1.0 - x2))
        val = pref * jnp.exp(earg)
        val = jnp.where(inside, val, jnp.float32(0.0))
        mol = (i0.astype(jnp.float32) * inv_atoms).astype(jnp.int32)
        plsc.addupdate_scatter(acc_v, [mol], val)

    pltpu.emit_pipeline(
        chunk_body,
        grid=(grid,),
        in_specs=[
            pl.BlockSpec((CHUNK,), lambda i: (i,)),
            pl.BlockSpec((CHUNK,), lambda i: (i + grid,)),
            pl.BlockSpec((CHUNK,), lambda i: (i,)),
        ],
        out_specs=[],
        core_axis_name=("c", "s"),
        dimension_semantics=(pltpu.PARALLEL,),
    )(a12_hbm, a12_hbm, d_hbm)

    wid = lax.axis_index("s") * 2 + lax.axis_index("c")
    pltpu.sync_copy(acc_v, out_hbm.at[wid])

  return kern


def kernel(species, energies, atom_index12, distances, exp_prefactor,
           distances_factor):
  n_mol = energies.shape[0]
  num_atoms = species.shape[1]
  n_edges = distances.shape[0]
  n_bins = 1024

  flat_species = species.reshape(-1).astype(jnp.int32)
  a12 = atom_index12.astype(jnp.int32).reshape(-1)
  pref16 = exp_prefactor.reshape(-1).astype(jnp.float32)
  dfac16 = (distances_factor.reshape(-1) * ANGSTROM2BOHR).astype(jnp.float32)

  kern = _srb_sc_kernel(n_edges, flat_species.shape[0], num_atoms, n_bins)
  partial = kern(flat_species, a12, distances.astype(jnp.float32),
                 pref16, dfac16)
  energies_out = energies + partial.sum(axis=0)[:n_mol]
  return species, energies_out


# speedup vs baseline: 1836.3369x; 1.3227x over previous
import dataclasses
import functools

import jax
import jax.numpy as jnp
import numpy as np
from jax import lax
from jax.experimental import pallas as pl
from jax.experimental.pallas import tpu as pltpu
from jax.experimental.pallas import tpu_sc as plsc

ANGSTROM2BOHR = 1.8897261258369282
CUTOFF = 5.2 * ANGSTROM2BOHR
LANES = 16
NUM_WORKERS = 32
CHUNK = 2560


def _srb_sc_kernel(n_edges, n_flat_atoms, num_atoms, n_bins):
  grid = n_edges // CHUNK
  mesh = plsc.VectorSubcoreMesh(core_axis_name="c", subcore_axis_name="s")

  cp = pltpu.CompilerParams()
  if "needs_layout_passes" in pltpu.CompilerParams.__dataclass_fields__:
    cp = dataclasses.replace(cp, needs_layout_passes=False)

  @functools.partial(
      pl.kernel,
      out_type=jax.ShapeDtypeStruct((NUM_WORKERS, n_bins), jnp.float32),
      mesh=mesh,
      compiler_params=cp,
      scratch_types=[
          pltpu.VMEM((n_flat_atoms,), jnp.int32),
          pltpu.VMEM((LANES,), jnp.float32),
          pltpu.VMEM((LANES,), jnp.float32),
          pltpu.VMEM((n_bins,), jnp.float32),
      ],
  )
  def kern(sp_hbm, a12_hbm, d_hbm, pref_hbm, dfac_hbm, out_hbm,
           sp_v, pref_v, dfac_v, acc_v):
    pltpu.sync_copy(sp_hbm, sp_v)
    pltpu.sync_copy(pref_hbm, pref_v)
    pltpu.sync_copy(dfac_hbm, dfac_v)

    @pl.loop(0, n_bins, step=LANES)
    def _(i):
      acc_v[pl.ds(i, LANES)] = jnp.zeros((LANES,), jnp.float32)

    inv_cutoff = jnp.float32(ANGSTROM2BOHR / CUTOFF)
    inv_atoms = jnp.float32(
        float(np.nextafter(np.float32(1.0 / num_atoms), np.float32(2.0))))

    def chunk_body(a_v, d_v):
      @plsc.parallel_loop(0, CHUNK, step=LANES, unroll=8)
      def _(c):
        i0 = a_v[0, pl.ds(c, LANES)]
        i1 = a_v[1, pl.ds(c, LANES)]
        dist = d_v[pl.ds(c, LANES)]
        s1 = plsc.load_gather(sp_v, [i0])
        s2 = plsc.load_gather(sp_v, [i1])
        k = s1 * 4 + s2
        pref = plsc.load_gather(pref_v, [k])
        dfac = plsc.load_gather(dfac_v, [k])
        x = dist * inv_cutoff
        x2 = x * x
        inside = x < 1.0
        earg = dfac * dist + (1.0 - 1.0 / (1.0 - x2))
        val = pref * jnp.exp(earg)
        val = jnp.where(inside, val, jnp.float32(0.0))
        mol = (i0.astype(jnp.float32) * inv_atoms).astype(jnp.int32)
        plsc.addupdate_scatter(acc_v, [mol], val)

    pltpu.emit_pipeline(
        chunk_body,
        grid=(grid,),
        in_specs=[
            pl.BlockSpec((2, CHUNK), lambda i: (0, i)),
            pl.BlockSpec((CHUNK,), lambda i: (i,)),
        ],
        out_specs=[],
        core_axis_name=("c", "s"),
        dimension_semantics=(pltpu.PARALLEL,),
    )(a12_hbm, d_hbm)

    wid = lax.axis_index("s") * 2 + lax.axis_index("c")
    pltpu.sync_copy(acc_v, out_hbm.at[wid])

  return kern


def kernel(species, energies, atom_index12, distances, exp_prefactor,
           distances_factor):
  n_mol = energies.shape[0]
  num_atoms = species.shape[1]
  n_edges = distances.shape[0]
  n_bins = 1024

  flat_species = species.reshape(-1).astype(jnp.int32)
  a12 = atom_index12.astype(jnp.int32)
  pref16 = exp_prefactor.reshape(-1).astype(jnp.float32)
  dfac16 = (distances_factor.reshape(-1) * ANGSTROM2BOHR).astype(jnp.float32)

  kern = _srb_sc_kernel(n_edges, flat_species.shape[0], num_atoms, n_bins)
  partial = kern(flat_species, a12, distances.astype(jnp.float32),
                 pref16, dfac16)
  energies_out = energies + partial.sum(axis=0)[:n_mol]
  return species, energies_out
